# baseline (speedup 1.0000x reference)
import functools

import jax
import jax.numpy as jnp
from jax import lax
from jax.experimental import pallas as pl
from jax.experimental.pallas import tpu as pltpu
from jax.experimental.pallas import tpu_sc as plsc

N = 10000
E = 320000
HID = 128
OUTD = 64

NC = 2
NS = 16
M = 10112
STRIPE = M // NS
B = 128
NB = 81
C_PER_TILE = NB * B
E_PER_CORE = NS * C_PER_TILE
ET_PAD = NC * E_PER_CORE
DUMMY_SRC = N
DUMMY_DST = 10050

BR = 632
NBLK = M // BR


_Z_FULL = STRIPE // B
_Z_TAIL = STRIPE % B


def _zero_stripe(zsrc, dst_sh, row0):
    def zcp(t, _):
        pltpu.sync_copy(zsrc, dst_sh.at[pl.ds(row0 + t * B, B)])
        return 0
    lax.fori_loop(0, _Z_FULL, zcp, 0)
    if _Z_TAIL:
        pltpu.sync_copy(zsrc.at[pl.ds(0, _Z_TAIL)],
                        dst_sh.at[pl.ds(row0 + _Z_FULL * B, _Z_TAIL)])


def _msg_body(h_hbm, pk3_hbm, zrows_hbm, tok_hbm, msg_out,
              pk_v, src0_v, dst0_v, src1_v, dst1_v,
              rows0_v, rows1_v, msg_sh, sem0, sem1, ssem0, ssem1):
    del tok_hbm
    cid = lax.axis_index("c")
    sid = lax.axis_index("s")
    row0 = sid * STRIPE
    w = cid * NS + sid

    def pre(t, _):
        sl = pl.ds(t * 8, 8)
        pltpu.sync_copy(pk3_hbm.at[w, sl], pk_v.at[sl])
        return 0
    lax.fori_loop(0, NB // 8, pre, 0)
    if NB % 8:
        sl = pl.ds((NB // 8) * 8, NB % 8)
        pltpu.sync_copy(pk3_hbm.at[w, sl], pk_v.at[sl])
    _zero_stripe(zrows_hbm, msg_sh, row0)
    plsc.subcore_barrier()

    def unpack(b, src_v, dst_v):
        for i in range(B // 16):
            v = pk_v[b, pl.ds(i * 16, 16)]
            src_v[pl.ds(i * 16, 16)] = v & 0xFFFF
            dst_v[pl.ds(i * 16, 16)] = lax.shift_right_logical(v, 16)

    def gather(rows_v, src_v, sem):
        pltpu.async_copy(h_hbm.at[src_v], rows_v, sem)

    def gwait(rows_v, src_v, sem):
        pltpu.make_async_copy(h_hbm.at[src_v], rows_v, sem).wait()

    def scat(rows_v, dst_v, ssem):
        pltpu.async_copy(rows_v, msg_sh.at[dst_v], ssem, add=True)

    def swait(rows_v, dst_v, ssem):
        pltpu.make_async_copy(rows_v, msg_sh.at[dst_v], ssem).wait()

    unpack(0, src0_v, dst0_v)
    gather(rows0_v, src0_v, sem0)
    unpack(1, src1_v, dst1_v)
    gather(rows1_v, src1_v, sem1)
    gwait(rows0_v, src0_v, sem0)
    scat(rows0_v, dst0_v, ssem0)

    def pair(t, _):
        b = 2 * t
        gwait(rows1_v, src1_v, sem1)
        scat(rows1_v, dst1_v, ssem1)

        @pl.when(b + 2 < NB)
        def _():
            swait(rows0_v, dst0_v, ssem0)
            unpack(b + 2, src0_v, dst0_v)
            gather(rows0_v, src0_v, sem0)
            gwait(rows0_v, src0_v, sem0)
            scat(rows0_v, dst0_v, ssem0)

        @pl.when(b + 3 < NB)
        def _():
            swait(rows1_v, dst1_v, ssem1)
            unpack(b + 3, src1_v, dst1_v)
            gather(rows1_v, src1_v, sem1)
        return 0
    lax.fori_loop(0, (NB - 1) // 2, pair, 0)
    swait(rows0_v, dst0_v, ssem0)
    swait(rows1_v, dst1_v, ssem1)

    plsc.subcore_barrier()
    pltpu.sync_copy(msg_sh.at[pl.ds(row0, STRIPE)],
                    msg_out.at[cid, pl.ds(row0, STRIPE)])


@functools.cache
def _agg():
    return pl.kernel(
        _msg_body,
        out_type=[jax.ShapeDtypeStruct((NC, M, HID), jnp.float32)],
        mesh=plsc.VectorSubcoreMesh(core_axis_name="c", subcore_axis_name="s"),
        compiler_params=pltpu.CompilerParams(needs_layout_passes=False),
        scratch_types=[
            pltpu.VMEM((NB, B), jnp.int32),
            pltpu.VMEM((B,), jnp.int32),
            pltpu.VMEM((B,), jnp.int32),
            pltpu.VMEM((B,), jnp.int32),
            pltpu.VMEM((B,), jnp.int32),
            pltpu.VMEM((B, HID), jnp.float32),
            pltpu.VMEM((B, HID), jnp.float32),
            pltpu.VMEM_SHARED((M, HID), jnp.float32),
            pltpu.SemaphoreType.DMA,
            pltpu.SemaphoreType.DMA,
            pltpu.SemaphoreType.DMA,
            pltpu.SemaphoreType.DMA,
        ],
    )


def _deg_body(dst3_hbm, deg_out, dsts_v, deg_v, red_v, deg8_v, stage_sh):
    cid = lax.axis_index("c")
    sid = lax.axis_index("s")
    w = cid * NS + sid

    def pre(t, _):
        sl = pl.ds(t * 8, 8)
        pltpu.sync_copy(dst3_hbm.at[w, sl], dsts_v.at[sl])
        return 0
    lax.fori_loop(0, NB // 8, pre, 0)
    if NB % 8:
        sl = pl.ds((NB // 8) * 8, NB % 8)
        pltpu.sync_copy(dst3_hbm.at[w, sl], dsts_v.at[sl])

    def zloop(r, _):
        deg_v[pl.ds(r * 16, 16)] = jnp.zeros((16,), jnp.float32)
        return 0
    lax.fori_loop(0, M // 16, zloop, 0)

    one16 = jnp.ones((16,), jnp.float32)

    def hist(b, _):
        for i in range(B // 16):
            idx = dsts_v[b, pl.ds(i * 16, 16)]
            plsc.addupdate_scatter(deg_v, [idx], one16)
        return 0
    lax.fori_loop(0, NB, hist, 0)

    pltpu.sync_copy(deg_v, stage_sh.at[sid])
    plsc.subcore_barrier()

    nchunk = M // 128
    for k in range((nchunk + NS - 1) // NS):
        c = k * NS + sid

        @pl.when(c < nchunk)
        def _():
            pltpu.sync_copy(stage_sh.at[:, pl.ds(c * 128, 128)], red_v)

            def rr(r, _):
                acc = red_v[0, pl.ds(r * 16, 16)]
                for t in range(1, NS):
                    acc = acc + red_v[t, pl.ds(r * 16, 16)]
                rows = lax.iota(jnp.int32, 16) + r * 16
                for cc in range(8):
                    plsc.store_scatter(
                        deg8_v, [rows, jnp.full((16,), cc, jnp.int32)], acc)
                return 0
            lax.fori_loop(0, 8, rr, 0)
            pltpu.sync_copy(deg8_v, deg_out.at[cid, pl.ds(c * 128, 128)])


@functools.cache
def _deg_count():
    return pl.kernel(
        _deg_body,
        out_type=[jax.ShapeDtypeStruct((NC, M, 8), jnp.float32)],
        mesh=plsc.VectorSubcoreMesh(core_axis_name="c", subcore_axis_name="s"),
        compiler_params=pltpu.CompilerParams(needs_layout_passes=False),
        scratch_types=[
            pltpu.VMEM((NB, B), jnp.int32),
            pltpu.VMEM((M,), jnp.float32),
            pltpu.VMEM((NS, 128), jnp.float32),
            pltpu.VMEM((128, 8), jnp.float32),
            pltpu.VMEM_SHARED((NS, M), jnp.float32),
        ],
    )


def _dense_a_body(msg_ref, deg0_ref, deg1_ref, h_ref, wl_ref, bl_ref, wr_ref,
                  h1_ref, s1_ref, s2_ref):
    i = pl.program_id(0)
    msg = msg_ref[0] + msg_ref[1]
    deg = deg0_ref[:, 0:1] + deg1_ref[:, 0:1]
    agg = msg * (1.0 / jnp.maximum(deg, 1.0))
    z = (jnp.dot(agg, wl_ref[...].T, preferred_element_type=jnp.float32)
         + bl_ref[...]
         + jnp.dot(h_ref[...], wr_ref[...].T,
                   preferred_element_type=jnp.float32))
    h1 = jnp.maximum(z, 0.0)
    h1_ref[...] = h1
    ridx = lax.broadcasted_iota(jnp.int32, (BR, 1), 0) + i * BR
    h1m = jnp.where(ridx < N, h1, 0.0)

    @pl.when(i == 0)
    def _():
        s1_ref[...] = jnp.zeros_like(s1_ref)
        s2_ref[...] = jnp.zeros_like(s2_ref)
    s1_ref[...] += jnp.sum(h1m, axis=0, keepdims=True)
    s2_ref[...] += jnp.sum(h1m * h1m, axis=0, keepdims=True)


_dense_a = pl.pallas_call(
    _dense_a_body,
    grid=(NBLK,),
    in_specs=[
        pl.BlockSpec((NC, BR, HID), lambda i: (0, i, 0)),
        pl.BlockSpec((BR, HID), lambda i: (i, 0)),
        pl.BlockSpec((BR, HID), lambda i: (i, 0)),
        pl.BlockSpec((BR, HID), lambda i: (i, 0)),
        pl.BlockSpec((HID, HID), lambda i: (0, 0)),
        pl.BlockSpec((1, HID), lambda i: (0, 0)),
        pl.BlockSpec((HID, HID), lambda i: (0, 0)),
    ],
    out_specs=[
        pl.BlockSpec((BR, HID), lambda i: (i, 0)),
        pl.BlockSpec((1, HID), lambda i: (0, 0)),
        pl.BlockSpec((1, HID), lambda i: (0, 0)),
    ],
    out_shape=[
        jax.ShapeDtypeStruct((M, HID), jnp.float32),
        jax.ShapeDtypeStruct((1, HID), jnp.float32),
        jax.ShapeDtypeStruct((1, HID), jnp.float32),
    ],
)


def _norm_block(h1_ref, s1_ref, s2_ref, gw_ref, gb_ref, gms_ref):
    m = s1_ref[...] * (1.0 / N)
    ex2 = s2_ref[...] * (1.0 / N)
    ms = gms_ref[...]
    var = ex2 - (2.0 * ms - ms * ms) * (m * m)
    scale = lax.rsqrt(var + 1e-5) * gw_ref[...]
    hc = h1_ref[...] - m * ms
    return jnp.maximum(hc * scale + gb_ref[...], 0.0)


def _dense_b_body(h1_ref, s1_ref, s2_ref, gw_ref, gb_ref, gms_ref, out_ref):
    out_ref[...] = _norm_block(h1_ref, s1_ref, s2_ref, gw_ref, gb_ref, gms_ref)


def _dense_b2_body(h1_ref, s1_ref, s2_ref, gw_ref, gb_ref, gms_ref,
                   wlin_ref, blin_ref, out_ref):
    g = _norm_block(h1_ref, s1_ref, s2_ref, gw_ref, gb_ref, gms_ref)
    out_ref[...] = (jnp.dot(g, wlin_ref[...].T,
                            preferred_element_type=jnp.float32)
                    + blin_ref[...])


_B_IN_SPECS = [
    pl.BlockSpec((BR, HID), lambda i: (i, 0)),
    pl.BlockSpec((1, HID), lambda i: (0, 0)),
    pl.BlockSpec((1, HID), lambda i: (0, 0)),
    pl.BlockSpec((1, HID), lambda i: (0, 0)),
    pl.BlockSpec((1, HID), lambda i: (0, 0)),
    pl.BlockSpec((1, HID), lambda i: (0, 0)),
]

_dense_b = pl.pallas_call(
    _dense_b_body,
    grid=(NBLK,),
    in_specs=_B_IN_SPECS,
    out_specs=pl.BlockSpec((BR, HID), lambda i: (i, 0)),
    out_shape=jax.ShapeDtypeStruct((M, HID), jnp.float32),
)

_dense_b2 = pl.pallas_call(
    _dense_b2_body,
    grid=(NBLK,),
    in_specs=_B_IN_SPECS + [
        pl.BlockSpec((OUTD, HID), lambda i: (0, 0)),
        pl.BlockSpec((1, OUTD), lambda i: (0, 0)),
    ],
    out_specs=pl.BlockSpec((BR, OUTD), lambda i: (i, 0)),
    out_shape=jax.ShapeDtypeStruct((M, OUTD), jnp.float32),
)


def kernel(x, edge_index, W1l, b1l, W1r, W2l, b2l, W2r,
           gn1_w, gn1_b, gn1_ms, gn2_w, gn2_b, gn2_ms, Wlin, blin):
    f32 = jnp.float32
    x_pad = jnp.pad(x, ((0, M - N), (0, 0)))
    loops = jnp.arange(N, dtype=jnp.int32)
    pad = ET_PAD - (E + N)
    src = jnp.concatenate([edge_index[0], loops,
                           jnp.full((pad,), DUMMY_SRC, jnp.int32)])
    dst = jnp.concatenate([edge_index[1], loops,
                           jnp.full((pad,), DUMMY_DST, jnp.int32)])
    pk = (src | (dst << 16)).reshape(NC * NS, NB, B)
    dst3 = dst.reshape(NC * NS, NB, B)

    zrows = jnp.zeros((B, HID), f32)
    (deg,) = _deg_count()(dst3)
    tok = deg[0, :8, 0]
    deg0 = jnp.tile(deg[0], (1, HID // 8))
    deg1 = jnp.tile(deg[1], (1, HID // 8))
    (msg1,) = _agg()(x_pad, pk, zrows, tok)
    h1a, s1, s2 = _dense_a(msg1, deg0, deg1, x_pad, W1l,
                           b1l.reshape(1, HID), W1r)
    h1 = _dense_b(h1a, s1, s2, gn1_w.reshape(1, HID), gn1_b.reshape(1, HID),
                  gn1_ms.reshape(1, HID))
    (msg2,) = _agg()(h1, pk, zrows, tok)
    h2a, t1, t2 = _dense_a(msg2, deg0, deg1, h1, W2l,
                           b2l.reshape(1, HID), W2r)
    out = _dense_b2(h2a, t1, t2, gn2_w.reshape(1, HID), gn2_b.reshape(1, HID),
                    gn2_ms.reshape(1, HID), Wlin, blin.reshape(1, OUTD))
    return out[:N]


# speedup vs baseline: 1.0971x; 1.0971x over previous
import functools

import jax
import jax.numpy as jnp
from jax import lax
from jax.experimental import pallas as pl
from jax.experimental.pallas import tpu as pltpu
from jax.experimental.pallas import tpu_sc as plsc

N = 10000
E = 320000
HID = 128
OUTD = 64

NC = 2
NS = 16
M = 10112
STRIPE = M // NS
B = 128
NB = 81
C_PER_TILE = NB * B
E_PER_CORE = NS * C_PER_TILE
ET_PAD = NC * E_PER_CORE
DUMMY_SRC = N
DUMMY_DST = 10050

BR = 632
NBLK = M // BR


_Z_FULL = STRIPE // B
_Z_TAIL = STRIPE % B


def _zero_stripe(zsrc, dst_sh, row0):
    def zcp(t, _):
        pltpu.sync_copy(zsrc, dst_sh.at[pl.ds(row0 + t * B, B)])
        return 0
    lax.fori_loop(0, _Z_FULL, zcp, 0)
    if _Z_TAIL:
        pltpu.sync_copy(zsrc.at[pl.ds(0, _Z_TAIL)],
                        dst_sh.at[pl.ds(row0 + _Z_FULL * B, _Z_TAIL)])


def _msg_body(h_hbm, pk3_hbm, zrows_hbm, tok_hbm, msg_out,
              pk_v, src0_v, dst0_v, src1_v, dst1_v,
              rows0_v, rows1_v, msg_sh, sem0, sem1):
    del tok_hbm
    cid = lax.axis_index("c")
    sid = lax.axis_index("s")
    row0 = sid * STRIPE
    w = cid * NS + sid

    def pre(t, _):
        sl = pl.ds(t * 8, 8)
        pltpu.sync_copy(pk3_hbm.at[w, sl], pk_v.at[sl])
        return 0
    lax.fori_loop(0, NB // 8, pre, 0)
    if NB % 8:
        sl = pl.ds((NB // 8) * 8, NB % 8)
        pltpu.sync_copy(pk3_hbm.at[w, sl], pk_v.at[sl])
    _zero_stripe(zrows_hbm, msg_sh, row0)
    plsc.subcore_barrier()

    def unpack(b, src_v, dst_v):
        for i in range(B // 16):
            v = pk_v[b, pl.ds(i * 16, 16)]
            src_v[pl.ds(i * 16, 16)] = v & 0xFFFF
            dst_v[pl.ds(i * 16, 16)] = lax.shift_right_logical(v, 16)

    def gather(rows_v, src_v, sem):
        pltpu.async_copy(h_hbm.at[src_v], rows_v, sem)

    def gwait(rows_v, src_v, sem):
        pltpu.make_async_copy(h_hbm.at[src_v], rows_v, sem).wait()

    def scat(rows_v, dst_v):
        pltpu.sync_copy(rows_v, msg_sh.at[dst_v], add=True)

    unpack(0, src0_v, dst0_v)
    gather(rows0_v, src0_v, sem0)

    def pair(t, _):
        b = 2 * t

        @pl.when(b + 1 < NB)
        def _():
            unpack(b + 1, src1_v, dst1_v)
            gather(rows1_v, src1_v, sem1)
        gwait(rows0_v, src0_v, sem0)
        scat(rows0_v, dst0_v)

        @pl.when(b + 2 < NB)
        def _():
            unpack(b + 2, src0_v, dst0_v)
            gather(rows0_v, src0_v, sem0)

        @pl.when(b + 1 < NB)
        def _():
            gwait(rows1_v, src1_v, sem1)
            scat(rows1_v, dst1_v)
        return 0
    lax.fori_loop(0, (NB + 1) // 2, pair, 0)

    plsc.subcore_barrier()
    pltpu.sync_copy(msg_sh.at[pl.ds(row0, STRIPE)],
                    msg_out.at[cid, pl.ds(row0, STRIPE)])


@functools.cache
def _agg():
    return pl.kernel(
        _msg_body,
        out_type=[jax.ShapeDtypeStruct((NC, M, HID), jnp.float32)],
        mesh=plsc.VectorSubcoreMesh(core_axis_name="c", subcore_axis_name="s"),
        compiler_params=pltpu.CompilerParams(needs_layout_passes=False),
        scratch_types=[
            pltpu.VMEM((NB, B), jnp.int32),
            pltpu.VMEM((B,), jnp.int32),
            pltpu.VMEM((B,), jnp.int32),
            pltpu.VMEM((B,), jnp.int32),
            pltpu.VMEM((B,), jnp.int32),
            pltpu.VMEM((B, HID), jnp.float32),
            pltpu.VMEM((B, HID), jnp.float32),
            pltpu.VMEM_SHARED((M, HID), jnp.float32),
            pltpu.SemaphoreType.DMA,
            pltpu.SemaphoreType.DMA,
        ],
    )


def _deg_body(dst3_hbm, deg_out, dsts_v, deg_v, red_v, deg8_v, stage_sh):
    cid = lax.axis_index("c")
    sid = lax.axis_index("s")
    w = cid * NS + sid

    def pre(t, _):
        sl = pl.ds(t * 8, 8)
        pltpu.sync_copy(dst3_hbm.at[w, sl], dsts_v.at[sl])
        return 0
    lax.fori_loop(0, NB // 8, pre, 0)
    if NB % 8:
        sl = pl.ds((NB // 8) * 8, NB % 8)
        pltpu.sync_copy(dst3_hbm.at[w, sl], dsts_v.at[sl])

    def zloop(r, _):
        deg_v[pl.ds(r * 16, 16)] = jnp.zeros((16,), jnp.float32)
        return 0
    lax.fori_loop(0, M // 16, zloop, 0)

    one16 = jnp.ones((16,), jnp.float32)

    def hist(b, _):
        for i in range(B // 16):
            idx = dsts_v[b, pl.ds(i * 16, 16)]
            plsc.addupdate_scatter(deg_v, [idx], one16)
        return 0
    lax.fori_loop(0, NB, hist, 0)

    pltpu.sync_copy(deg_v, stage_sh.at[sid])
    plsc.subcore_barrier()

    nchunk = M // 128
    for k in range((nchunk + NS - 1) // NS):
        c = k * NS + sid

        @pl.when(c < nchunk)
        def _():
            pltpu.sync_copy(stage_sh.at[:, pl.ds(c * 128, 128)], red_v)

            def rr(r, _):
                acc = red_v[0, pl.ds(r * 16, 16)]
                for t in range(1, NS):
                    acc = acc + red_v[t, pl.ds(r * 16, 16)]
                rows = lax.iota(jnp.int32, 16) + r * 16
                for cc in range(8):
                    plsc.store_scatter(
                        deg8_v, [rows, jnp.full((16,), cc, jnp.int32)], acc)
                return 0
            lax.fori_loop(0, 8, rr, 0)
            pltpu.sync_copy(deg8_v, deg_out.at[cid, pl.ds(c * 128, 128)])


@functools.cache
def _deg_count():
    return pl.kernel(
        _deg_body,
        out_type=[jax.ShapeDtypeStruct((NC, M, 8), jnp.float32)],
        mesh=plsc.VectorSubcoreMesh(core_axis_name="c", subcore_axis_name="s"),
        compiler_params=pltpu.CompilerParams(needs_layout_passes=False),
        scratch_types=[
            pltpu.VMEM((NB, B), jnp.int32),
            pltpu.VMEM((M,), jnp.float32),
            pltpu.VMEM((NS, 128), jnp.float32),
            pltpu.VMEM((128, 8), jnp.float32),
            pltpu.VMEM_SHARED((NS, M), jnp.float32),
        ],
    )


def _dense_a_body(msg_ref, deg0_ref, deg1_ref, h_ref, wl_ref, bl_ref, wr_ref,
                  h1_ref, s1_ref, s2_ref):
    i = pl.program_id(0)
    msg = msg_ref[0] + msg_ref[1]
    deg = deg0_ref[:, 0:1] + deg1_ref[:, 0:1]
    agg = msg * (1.0 / jnp.maximum(deg, 1.0))
    z = (jnp.dot(agg, wl_ref[...].T, preferred_element_type=jnp.float32)
         + bl_ref[...]
         + jnp.dot(h_ref[...], wr_ref[...].T,
                   preferred_element_type=jnp.float32))
    h1 = jnp.maximum(z, 0.0)
    h1_ref[...] = h1
    ridx = lax.broadcasted_iota(jnp.int32, (BR, 1), 0) + i * BR
    h1m = jnp.where(ridx < N, h1, 0.0)

    @pl.when(i == 0)
    def _():
        s1_ref[...] = jnp.zeros_like(s1_ref)
        s2_ref[...] = jnp.zeros_like(s2_ref)
    s1_ref[...] += jnp.sum(h1m, axis=0, keepdims=True)
    s2_ref[...] += jnp.sum(h1m * h1m, axis=0, keepdims=True)


_dense_a = pl.pallas_call(
    _dense_a_body,
    grid=(NBLK,),
    in_specs=[
        pl.BlockSpec((NC, BR, HID), lambda i: (0, i, 0)),
        pl.BlockSpec((BR, HID), lambda i: (i, 0)),
        pl.BlockSpec((BR, HID), lambda i: (i, 0)),
        pl.BlockSpec((BR, HID), lambda i: (i, 0)),
        pl.BlockSpec((HID, HID), lambda i: (0, 0)),
        pl.BlockSpec((1, HID), lambda i: (0, 0)),
        pl.BlockSpec((HID, HID), lambda i: (0, 0)),
    ],
    out_specs=[
        pl.BlockSpec((BR, HID), lambda i: (i, 0)),
        pl.BlockSpec((1, HID), lambda i: (0, 0)),
        pl.BlockSpec((1, HID), lambda i: (0, 0)),
    ],
    out_shape=[
        jax.ShapeDtypeStruct((M, HID), jnp.float32),
        jax.ShapeDtypeStruct((1, HID), jnp.float32),
        jax.ShapeDtypeStruct((1, HID), jnp.float32),
    ],
)


def _norm_block(h1_ref, s1_ref, s2_ref, gw_ref, gb_ref, gms_ref):
    m = s1_ref[...] * (1.0 / N)
    ex2 = s2_ref[...] * (1.0 / N)
    ms = gms_ref[...]
    var = ex2 - (2.0 * ms - ms * ms) * (m * m)
    scale = lax.rsqrt(var + 1e-5) * gw_ref[...]
    hc = h1_ref[...] - m * ms
    return jnp.maximum(hc * scale + gb_ref[...], 0.0)


def _dense_b_body(h1_ref, s1_ref, s2_ref, gw_ref, gb_ref, gms_ref, out_ref):
    out_ref[...] = _norm_block(h1_ref, s1_ref, s2_ref, gw_ref, gb_ref, gms_ref)


def _dense_b2_body(h1_ref, s1_ref, s2_ref, gw_ref, gb_ref, gms_ref,
                   wlin_ref, blin_ref, out_ref):
    g = _norm_block(h1_ref, s1_ref, s2_ref, gw_ref, gb_ref, gms_ref)
    out_ref[...] = (jnp.dot(g, wlin_ref[...].T,
                            preferred_element_type=jnp.float32)
                    + blin_ref[...])


_B_IN_SPECS = [
    pl.BlockSpec((BR, HID), lambda i: (i, 0)),
    pl.BlockSpec((1, HID), lambda i: (0, 0)),
    pl.BlockSpec((1, HID), lambda i: (0, 0)),
    pl.BlockSpec((1, HID), lambda i: (0, 0)),
    pl.BlockSpec((1, HID), lambda i: (0, 0)),
    pl.BlockSpec((1, HID), lambda i: (0, 0)),
]

_dense_b = pl.pallas_call(
    _dense_b_body,
    grid=(NBLK,),
    in_specs=_B_IN_SPECS,
    out_specs=pl.BlockSpec((BR, HID), lambda i: (i, 0)),
    out_shape=jax.ShapeDtypeStruct((M, HID), jnp.float32),
)

_dense_b2 = pl.pallas_call(
    _dense_b2_body,
    grid=(NBLK,),
    in_specs=_B_IN_SPECS + [
        pl.BlockSpec((OUTD, HID), lambda i: (0, 0)),
        pl.BlockSpec((1, OUTD), lambda i: (0, 0)),
    ],
    out_specs=pl.BlockSpec((BR, OUTD), lambda i: (i, 0)),
    out_shape=jax.ShapeDtypeStruct((M, OUTD), jnp.float32),
)


def kernel(x, edge_index, W1l, b1l, W1r, W2l, b2l, W2r,
           gn1_w, gn1_b, gn1_ms, gn2_w, gn2_b, gn2_ms, Wlin, blin):
    f32 = jnp.float32
    x_pad = jnp.pad(x, ((0, M - N), (0, 0)))
    loops = jnp.arange(N, dtype=jnp.int32)
    pad = ET_PAD - (E + N)
    src = jnp.concatenate([edge_index[0], loops,
                           jnp.full((pad,), DUMMY_SRC, jnp.int32)])
    dst = jnp.concatenate([edge_index[1], loops,
                           jnp.full((pad,), DUMMY_DST, jnp.int32)])
    pk = (src | (dst << 16)).reshape(NC * NS, NB, B)
    dst3 = dst.reshape(NC * NS, NB, B)

    zrows = jnp.zeros((B, HID), f32)
    (deg,) = _deg_count()(dst3)
    tok = deg[0, :8, 0]
    deg0 = jnp.tile(deg[0], (1, HID // 8))
    deg1 = jnp.tile(deg[1], (1, HID // 8))
    (msg1,) = _agg()(x_pad, pk, zrows, tok)
    h1a, s1, s2 = _dense_a(msg1, deg0, deg1, x_pad, W1l,
                           b1l.reshape(1, HID), W1r)
    h1 = _dense_b(h1a, s1, s2, gn1_w.reshape(1, HID), gn1_b.reshape(1, HID),
                  gn1_ms.reshape(1, HID))
    (msg2,) = _agg()(h1, pk, zrows, tok)
    h2a, t1, t2 = _dense_a(msg2, deg0, deg1, h1, W2l,
                           b2l.reshape(1, HID), W2r)
    out = _dense_b2(h2a, t1, t2, gn2_w.reshape(1, HID), gn2_b.reshape(1, HID),
                    gn2_ms.reshape(1, HID), Wlin, blin.reshape(1, OUTD))
    return out[:N]
